# Initial kernel scaffold; baseline (speedup 1.0000x reference)
import jax
import jax.numpy as jnp
from jax.experimental import pallas as pl


def kernel(a, b, emb_a, emb_b, W_comp, b_comp, W_out, b_out):
    raise NotImplementedError("write your pallas kernel here")


# speedup vs baseline: 4.5146x; 4.5146x over previous
import functools

import jax
import jax.numpy as jnp
from jax import lax
from jax.experimental import pallas as pl
from jax.experimental.pallas import tpu as pltpu
from jax.experimental.pallas import tpu_sc as plsc

N_HID = 60
N_SYM = 26
N_PAIR = N_SYM * N_SYM
TBL_PAD = 688
BATCH = 16384


def _table_body(emb_a_ref, emb_b_ref, wat_ref, wbt_ref, bc_ref, wo_ref,
                bo_ref, out_ref):
    ea = jnp.maximum(emb_a_ref[...], 0.0)
    eb = jnp.maximum(emb_b_ref[...], 0.0)
    A = jnp.dot(ea, wat_ref[...], preferred_element_type=jnp.float32)
    B = jnp.dot(eb, wbt_ref[...], preferred_element_type=jnp.float32)

    p_idx = lax.broadcasted_iota(jnp.int32, (TBL_PAD, N_SYM), 0) // N_SYM
    q_idx = lax.broadcasted_iota(jnp.int32, (TBL_PAD, N_SYM), 0) % N_SYM
    sym = lax.broadcasted_iota(jnp.int32, (TBL_PAD, N_SYM), 1)
    ph = jnp.where(p_idx == sym, 1.0, 0.0)
    qh = jnp.where(q_idx == sym, 1.0, 0.0)

    h = (jnp.dot(ph, A, preferred_element_type=jnp.float32)
         + jnp.dot(qh, B, preferred_element_type=jnp.float32)
         + bc_ref[...])
    h = jnp.maximum(h, 0.0)
    t = jnp.sum(h * wo_ref[...], axis=1, keepdims=True) + bo_ref[...]
    out_ref[...] = t


def _build_table(emb_a, emb_b, wat, wbt, b_comp, w_out, b_out):
    return pl.pallas_call(
        _table_body,
        out_shape=jax.ShapeDtypeStruct((TBL_PAD, 1), jnp.float32),
    )(emb_a, emb_b, wat, wbt, b_comp, w_out, b_out)


_NCORES = 2
_NSUB = 16
_NW = _NCORES * _NSUB
_PER_W = BATCH // _NW
_LANES = 16


def _gather_body(tbl_hbm, a_hbm, b_hbm, out_hbm, tbl_v, a_v, b_v, out_v):
    wid = lax.axis_index("s") * _NCORES + lax.axis_index("c")
    base = wid * _PER_W
    pltpu.sync_copy(tbl_hbm, tbl_v)
    pltpu.sync_copy(a_hbm.at[pl.ds(base, _PER_W)], a_v)
    pltpu.sync_copy(b_hbm.at[pl.ds(base, _PER_W)], b_v)
    for i in range(_PER_W // _LANES):
        sl = pl.ds(i * _LANES, _LANES)
        idx = a_v[sl] * N_SYM + b_v[sl]
        out_v[sl] = plsc.load_gather(tbl_v, [idx])
    pltpu.sync_copy(out_v, out_hbm.at[pl.ds(base, _PER_W)])


@functools.lru_cache(maxsize=1)
def _make_gather():
    return pl.kernel(
        _gather_body,
        out_type=jax.ShapeDtypeStruct((BATCH,), jnp.float32),
        mesh=plsc.VectorSubcoreMesh(core_axis_name="c", subcore_axis_name="s",
                                    num_cores=_NCORES, num_subcores=_NSUB),
        compiler_params=pltpu.CompilerParams(needs_layout_passes=False),
        scratch_types=[
            pltpu.VMEM((TBL_PAD,), jnp.float32),
            pltpu.VMEM((_PER_W,), jnp.int32),
            pltpu.VMEM((_PER_W,), jnp.int32),
            pltpu.VMEM((_PER_W,), jnp.float32),
        ],
    )


@jax.jit
def kernel(a, b, emb_a, emb_b, W_comp, b_comp, W_out, b_out):
    wat = W_comp[:, :N_HID].T
    wbt = W_comp[:, N_HID:].T
    table = _build_table(emb_a, emb_b, wat, wbt,
                         b_comp.reshape(1, N_HID), W_out,
                         b_out.reshape(1, 1))
    out = _make_gather()(table.reshape(TBL_PAD), a, b)
    return out.reshape(BATCH, 1)


# speedup vs baseline: 5.1580x; 1.1425x over previous
import functools

import jax
import jax.numpy as jnp
from jax import lax
from jax.experimental import pallas as pl
from jax.experimental.pallas import tpu as pltpu
from jax.experimental.pallas import tpu_sc as plsc

N_HID = 60
N_SYM = 26
N_PAIR = N_SYM * N_SYM
TBL_PAD = 688
BATCH = 16384


def _table_body(emb_a_ref, emb_b_ref, wc_ref, bc_ref, wo_ref, bo_ref,
                out_ref):
    ea = jnp.maximum(emb_a_ref[...], 0.0)
    eb = jnp.maximum(emb_b_ref[...], 0.0)

    p_idx = lax.broadcasted_iota(jnp.int32, (TBL_PAD, N_SYM), 0) // N_SYM
    q_idx = lax.broadcasted_iota(jnp.int32, (TBL_PAD, N_SYM), 0) % N_SYM
    sym = lax.broadcasted_iota(jnp.int32, (TBL_PAD, N_SYM), 1)
    ph = jnp.where(p_idx == sym, 1.0, 0.0)
    qh = jnp.where(q_idx == sym, 1.0, 0.0)

    cat = jnp.concatenate(
        (jnp.dot(ph, ea, preferred_element_type=jnp.float32),
         jnp.dot(qh, eb, preferred_element_type=jnp.float32)),
        axis=1)
    h = lax.dot_general(cat, wc_ref[...], (((1,), (1,)), ((), ())),
                        preferred_element_type=jnp.float32)
    h = jnp.maximum(h + bc_ref[...], 0.0)
    t = jnp.sum(h * wo_ref[...], axis=1, keepdims=True) + bo_ref[...]
    out_ref[...] = t


def _build_table(emb_a, emb_b, w_comp, b_comp, w_out, b_out):
    return pl.pallas_call(
        _table_body,
        out_shape=jax.ShapeDtypeStruct((TBL_PAD, 1), jnp.float32),
    )(emb_a, emb_b, w_comp, b_comp, w_out, b_out)


_NCORES = 2
_NSUB = 16
_NW = _NCORES * _NSUB
_PER_W = BATCH // _NW
_LANES = 16


def _gather_body(tbl_hbm, a_hbm, b_hbm, out_hbm, tbl_v, a_v, b_v, out_v,
                 sem_t, sem_a, sem_b):
    wid = lax.axis_index("s") * _NCORES + lax.axis_index("c")
    base = wid * _PER_W
    ct = pltpu.async_copy(tbl_hbm, tbl_v, sem_t)
    ca = pltpu.async_copy(a_hbm.at[pl.ds(base, _PER_W)], a_v, sem_a)
    cb = pltpu.async_copy(b_hbm.at[pl.ds(base, _PER_W)], b_v, sem_b)
    ct.wait()
    ca.wait()
    cb.wait()
    for i in range(_PER_W // _LANES):
        sl = pl.ds(i * _LANES, _LANES)
        idx = a_v[sl] * N_SYM + b_v[sl]
        out_v[sl] = plsc.load_gather(tbl_v, [idx])
    pltpu.sync_copy(out_v, out_hbm.at[pl.ds(base, _PER_W)])


@functools.lru_cache(maxsize=1)
def _make_gather():
    return pl.kernel(
        _gather_body,
        out_type=jax.ShapeDtypeStruct((BATCH,), jnp.float32),
        mesh=plsc.VectorSubcoreMesh(core_axis_name="c", subcore_axis_name="s",
                                    num_cores=_NCORES, num_subcores=_NSUB),
        compiler_params=pltpu.CompilerParams(needs_layout_passes=False),
        scratch_types=[
            pltpu.VMEM((TBL_PAD,), jnp.float32),
            pltpu.VMEM((_PER_W,), jnp.int32),
            pltpu.VMEM((_PER_W,), jnp.int32),
            pltpu.VMEM((_PER_W,), jnp.float32),
            pltpu.SemaphoreType.DMA,
            pltpu.SemaphoreType.DMA,
            pltpu.SemaphoreType.DMA,
        ],
    )


@jax.jit
def kernel(a, b, emb_a, emb_b, W_comp, b_comp, W_out, b_out):
    table = _build_table(emb_a, emb_b, W_comp,
                         b_comp.reshape(1, N_HID), W_out,
                         b_out.reshape(1, 1))
    out = _make_gather()(table.reshape(TBL_PAD), a, b)
    return out.reshape(BATCH, 1)


# speedup vs baseline: 5.6260x; 1.0907x over previous
import functools

import jax
import jax.numpy as jnp
from jax import lax
from jax.experimental import pallas as pl
from jax.experimental.pallas import tpu as pltpu
from jax.experimental.pallas import tpu_sc as plsc

N_HID = 60
N_SYM = 26
N_PAIR = N_SYM * N_SYM
TBL_PAD = 688
BATCH = 16384


def _table_body(emb_a_ref, emb_b_ref, wc_ref, bc_ref, wo_ref, bo_ref,
                out_ref):
    ea = jnp.maximum(emb_a_ref[...], 0.0)
    eb = jnp.maximum(emb_b_ref[...], 0.0)

    p_idx = lax.broadcasted_iota(jnp.int32, (TBL_PAD, N_SYM), 0) // N_SYM
    q_idx = lax.broadcasted_iota(jnp.int32, (TBL_PAD, N_SYM), 0) % N_SYM
    sym = lax.broadcasted_iota(jnp.int32, (TBL_PAD, N_SYM), 1)
    ph = jnp.where(p_idx == sym, 1.0, 0.0)
    qh = jnp.where(q_idx == sym, 1.0, 0.0)

    cat = jnp.concatenate(
        (jnp.dot(ph, ea, preferred_element_type=jnp.float32),
         jnp.dot(qh, eb, preferred_element_type=jnp.float32)),
        axis=1)
    h = lax.dot_general(cat, wc_ref[...], (((1,), (1,)), ((), ())),
                        preferred_element_type=jnp.float32)
    h = jnp.maximum(h + bc_ref[...], 0.0)
    t = jnp.sum(h * wo_ref[...], axis=1, keepdims=True) + bo_ref[...]
    out_ref[...] = t


def _build_table(emb_a, emb_b, w_comp, b_comp, w_out, b_out):
    return pl.pallas_call(
        _table_body,
        out_shape=jax.ShapeDtypeStruct((TBL_PAD, 1), jnp.float32),
    )(emb_a, emb_b, w_comp, b_comp, w_out, b_out)


_NCORES = 1
_NSUB = 16
_NW = _NCORES * _NSUB
_PER_W = BATCH // _NW
_LANES = 16


def _gather_body(tbl_hbm, a_hbm, b_hbm, out_hbm, tbl_v, a_v, b_v, out_v,
                 sem_t, sem_a, sem_b):
    wid = lax.axis_index("s")
    base = wid * _PER_W
    ct = pltpu.async_copy(tbl_hbm, tbl_v, sem_t)
    ca = pltpu.async_copy(a_hbm.at[pl.ds(base, _PER_W)], a_v, sem_a)
    cb = pltpu.async_copy(b_hbm.at[pl.ds(base, _PER_W)], b_v, sem_b)
    ct.wait()
    ca.wait()
    cb.wait()
    for i in range(_PER_W // _LANES):
        sl = pl.ds(i * _LANES, _LANES)
        idx = a_v[sl] * N_SYM + b_v[sl]
        out_v[sl] = plsc.load_gather(tbl_v, [idx])
    pltpu.sync_copy(out_v, out_hbm.at[pl.ds(base, _PER_W)])


@functools.lru_cache(maxsize=1)
def _make_gather():
    return pl.kernel(
        _gather_body,
        out_type=jax.ShapeDtypeStruct((BATCH,), jnp.float32),
        mesh=plsc.VectorSubcoreMesh(core_axis_name="c", subcore_axis_name="s",
                                    num_cores=_NCORES, num_subcores=_NSUB),
        compiler_params=pltpu.CompilerParams(needs_layout_passes=False),
        scratch_types=[
            pltpu.VMEM((TBL_PAD,), jnp.float32),
            pltpu.VMEM((_PER_W,), jnp.int32),
            pltpu.VMEM((_PER_W,), jnp.int32),
            pltpu.VMEM((_PER_W,), jnp.float32),
            pltpu.SemaphoreType.DMA,
            pltpu.SemaphoreType.DMA,
            pltpu.SemaphoreType.DMA,
        ],
    )


@jax.jit
def kernel(a, b, emb_a, emb_b, W_comp, b_comp, W_out, b_out):
    table = _build_table(emb_a, emb_b, W_comp,
                         b_comp.reshape(1, N_HID), W_out,
                         b_out.reshape(1, 1))
    out = _make_gather()(table.reshape(TBL_PAD), a, b)
    return out.reshape(BATCH, 1)
